# Initial kernel scaffold; baseline (speedup 1.0000x reference)
import jax
import jax.numpy as jnp
from jax.experimental import pallas as pl


def kernel(x1, edge_index1, x2, edge_index2, W1, b1, W2, b2):
    raise NotImplementedError("write your pallas kernel here")


# speedup vs baseline: 6.7020x; 6.7020x over previous
import functools

import jax
import jax.numpy as jnp
from jax import lax
from jax.experimental import pallas as pl
from jax.experimental.pallas import tpu as pltpu
from jax.experimental.pallas import tpu_sc as plsc

N = 10000
E = 320000
D = 128
NC, NS, L = 2, 16, 16
NW = NC * NS
TILE = 128
TW = 80
EP = NW * TW * TILE
ROWS_T = EP // TILE
R = 10240
RPS = R // NS
DUMMY = R - 1

_mesh = plsc.VectorSubcoreMesh(
    core_axis_name="c", subcore_axis_name="s", num_cores=NC, num_subcores=NS)


_DEG_OUT = tuple(jax.ShapeDtypeStruct((R,), jnp.float32) for _ in range(4))


@functools.partial(
    pl.kernel,
    out_type=_DEG_OUT,
    mesh=_mesh,
    scratch_types=[
        pltpu.VMEM((TW, TILE), jnp.int32),
        pltpu.VMEM((TILE,), jnp.float32),
        pltpu.VMEM_SHARED((R,), jnp.float32),
        pltpu.VMEM_SHARED((R,), jnp.float32),
    ],
)
def _deg_kernel(dst1_hbm, dst2_hbm, zeros1_hbm, ones_hbm,
                o1c0, o1c1, o2c0, o2c1,
                idx_v, ones_v, acc1_sh, acc2_sh):
    c = lax.axis_index("c")
    s = lax.axis_index("s")
    wbase = (c * NS + s) * TW

    pltpu.sync_copy(ones_hbm, ones_v)
    pltpu.sync_copy(zeros1_hbm, acc1_sh.at[pl.ds(s * RPS, RPS)])
    pltpu.sync_copy(zeros1_hbm, acc2_sh.at[pl.ds(s * RPS, RPS)])

    plsc.subcore_barrier()

    for dst_hbm, acc_sh in ((dst1_hbm, acc1_sh), (dst2_hbm, acc2_sh)):
        pltpu.sync_copy(dst_hbm.at[pl.ds(wbase, TW)], idx_v)

        @pl.loop(0, TW)
        def _(j):
            pltpu.sync_copy(ones_v, acc_sh.at[idx_v.at[j]], add=True)

    plsc.subcore_barrier()
    sl = pl.ds(s * RPS, RPS)

    @pl.when(c == 0)
    def _():
        pltpu.sync_copy(acc1_sh.at[sl], o1c0.at[sl])
        pltpu.sync_copy(acc2_sh.at[sl], o2c0.at[sl])

    @pl.when(c == 1)
    def _():
        pltpu.sync_copy(acc1_sh.at[sl], o1c1.at[sl])
        pltpu.sync_copy(acc2_sh.at[sl], o2c1.at[sl])


@functools.partial(
    pl.kernel,
    out_type=jax.ShapeDtypeStruct((NC, R, D), jnp.float32),
    mesh=_mesh,
    scratch_types=[
        pltpu.VMEM((TW, TILE), jnp.int32),
        pltpu.VMEM((TW, TILE), jnp.int32),
        pltpu.VMEM((TILE, D), jnp.float32),
        pltpu.VMEM_SHARED((R, D), jnp.float32),
    ],
)
def _agg_kernel(g_hbm, src_hbm, dst_hbm, zeros2_hbm, out_hbm,
                src_v, dst_v, buf_v, acc_sh):
    c = lax.axis_index("c")
    s = lax.axis_index("s")
    wbase = (c * NS + s) * TW

    pltpu.sync_copy(src_hbm.at[pl.ds(wbase, TW)], src_v)
    pltpu.sync_copy(dst_hbm.at[pl.ds(wbase, TW)], dst_v)
    pltpu.sync_copy(zeros2_hbm, acc_sh.at[pl.ds(s * RPS, RPS)])

    plsc.subcore_barrier()

    @pl.loop(0, TW)
    def _(j):
        pltpu.sync_copy(g_hbm.at[src_v.at[j]], buf_v)
        pltpu.sync_copy(buf_v, acc_sh.at[dst_v.at[j]], add=True)

    plsc.subcore_barrier()
    for k in range(RPS // TILE):
        sl = pl.ds(s * RPS + k * TILE, TILE)
        pltpu.sync_copy(acc_sh.at[sl], out_hbm.at[c, sl])


_HI = lax.Precision.HIGHEST


def _mm0_body(x1_ref, x2_ref, w_ref, h1_ref, h2_ref):
    h1_ref[...] = jnp.dot(x1_ref[...], w_ref[...],
                          preferred_element_type=jnp.float32, precision=_HI)
    h2_ref[...] = jnp.dot(x2_ref[...], w_ref[...],
                          preferred_element_type=jnp.float32, precision=_HI)


def _dinv_of(dpa_ref, dpb_ref):
    deg = dpa_ref[0:N, :] + dpb_ref[0:N, :] + 1.0
    return lax.rsqrt(jnp.maximum(deg, 1e-12))


def _acc_sum(acc_ref):
    return acc_ref[0, 0:N, :] + acc_ref[1, 0:N, :]


def _scale_body(dpa_ref, dpb_ref, h_ref, g_ref):
    g_ref[...] = h_ref[...] * _dinv_of(dpa_ref, dpb_ref)


def _layer2_body(dpa_ref, dpb_ref, acc_ref, g1_ref, w2_ref, b1_ref, g2_ref):
    dinv = _dinv_of(dpa_ref, dpb_ref)
    pre = (_acc_sum(acc_ref) + g1_ref[...]) * dinv + b1_ref[...]
    a = jnp.maximum(pre, 0.0)
    h2 = jnp.dot(a, w2_ref[...],
                 preferred_element_type=jnp.float32, precision=_HI)
    g2_ref[...] = h2 * dinv


def _znorm_body(dpa_ref, dpb_ref, acc_ref, g2_ref, b2_ref, z_ref):
    dinv = _dinv_of(dpa_ref, dpb_ref)
    o = (_acc_sum(acc_ref) + g2_ref[...]) * dinv + b2_ref[...]
    mu = jnp.mean(o, axis=0, keepdims=True)
    cen = o - mu
    var = jnp.sum(cen * cen, axis=0, keepdims=True) * (1.0 / (N - 1))
    z_ref[...] = cen * lax.rsqrt(var)


_f32 = lambda *shape: jax.ShapeDtypeStruct(shape, jnp.float32)

_mm0 = pl.pallas_call(_mm0_body, out_shape=(_f32(N, D), _f32(N, D)))
_scale = pl.pallas_call(_scale_body, out_shape=_f32(N, D))
_layer2 = pl.pallas_call(_layer2_body, out_shape=_f32(N, D))
_znorm = pl.pallas_call(_znorm_body, out_shape=_f32(N, D))


def _edge_tiles(edge_index):
    pad = EP - E
    src = jnp.concatenate(
        [edge_index[0], jnp.zeros((pad,), jnp.int32)]).reshape(ROWS_T, TILE)
    dst = jnp.concatenate(
        [edge_index[1], jnp.full((pad,), DUMMY, jnp.int32)]).reshape(ROWS_T, TILE)
    return src, dst


def kernel(x1, edge_index1, x2, edge_index2, W1, b1, W2, b2):
    src1, dst1 = _edge_tiles(edge_index1)
    src2, dst2 = _edge_tiles(edge_index2)
    b1r = b1.reshape(1, D)
    b2r = b2.reshape(1, D)

    zeros1 = jnp.zeros((RPS,), jnp.float32)
    zeros2 = jnp.zeros((RPS, D), jnp.float32)
    ones = jnp.ones((TILE,), jnp.float32)

    d1c0, d1c1, d2c0, d2c1 = _deg_kernel(dst1, dst2, zeros1, ones)
    dp1 = (d1c0.reshape(R, 1), d1c1.reshape(R, 1))
    dp2 = (d2c0.reshape(R, 1), d2c1.reshape(R, 1))

    h11, h12 = _mm0(x1, x2, W1)

    zs = []
    for dp, h1, src, dst in ((dp1, h11, src1, dst1), (dp2, h12, src2, dst2)):
        g1 = _scale(*dp, h1)
        acc1 = _agg_kernel(g1, src, dst, zeros2)
        g2 = _layer2(*dp, acc1, g1, W2, b1r)
        acc2 = _agg_kernel(g2, src, dst, zeros2)
        zs.append(_znorm(*dp, acc2, g2, b2r))
    return (zs[0], zs[1])


# speedup vs baseline: 6.8661x; 1.0245x over previous
import functools

import jax
import jax.numpy as jnp
from jax import lax
from jax.experimental import pallas as pl
from jax.experimental.pallas import tpu as pltpu
from jax.experimental.pallas import tpu_sc as plsc

N = 10000
E = 320000
D = 128
NC, NS, L = 2, 16, 16
NW = NC * NS
TILE = 128
TW = 80
EP = NW * TW * TILE
ROWS_T = EP // TILE
R = 10240
RPS = R // NS
DUMMY = R - 1

_mesh = plsc.VectorSubcoreMesh(
    core_axis_name="c", subcore_axis_name="s", num_cores=NC, num_subcores=NS)


_DEG_OUT = tuple(jax.ShapeDtypeStruct((R,), jnp.float32) for _ in range(4))


@functools.partial(
    pl.kernel,
    out_type=_DEG_OUT,
    mesh=_mesh,
    scratch_types=[
        pltpu.VMEM((TW, TILE), jnp.int32),
        pltpu.VMEM((TILE,), jnp.float32),
        pltpu.VMEM_SHARED((R,), jnp.float32),
        pltpu.VMEM_SHARED((R,), jnp.float32),
    ],
)
def _deg_kernel(dst1_hbm, dst2_hbm, zeros1_hbm, ones_hbm,
                o1c0, o1c1, o2c0, o2c1,
                idx_v, ones_v, acc1_sh, acc2_sh):
    c = lax.axis_index("c")
    s = lax.axis_index("s")
    wbase = (c * NS + s) * TW

    pltpu.sync_copy(ones_hbm, ones_v)
    pltpu.sync_copy(zeros1_hbm, acc1_sh.at[pl.ds(s * RPS, RPS)])
    pltpu.sync_copy(zeros1_hbm, acc2_sh.at[pl.ds(s * RPS, RPS)])

    plsc.subcore_barrier()

    for dst_hbm, acc_sh in ((dst1_hbm, acc1_sh), (dst2_hbm, acc2_sh)):
        pltpu.sync_copy(dst_hbm.at[pl.ds(wbase, TW)], idx_v)

        @pl.loop(0, TW)
        def _(j):
            pltpu.sync_copy(ones_v, acc_sh.at[idx_v.at[j]], add=True)

    plsc.subcore_barrier()
    sl = pl.ds(s * RPS, RPS)

    @pl.when(c == 0)
    def _():
        pltpu.sync_copy(acc1_sh.at[sl], o1c0.at[sl])
        pltpu.sync_copy(acc2_sh.at[sl], o2c0.at[sl])

    @pl.when(c == 1)
    def _():
        pltpu.sync_copy(acc1_sh.at[sl], o1c1.at[sl])
        pltpu.sync_copy(acc2_sh.at[sl], o2c1.at[sl])


NB = 2
TH = TW // 2


@functools.partial(
    pl.kernel,
    out_type=jax.ShapeDtypeStruct((NC, R, D), jnp.float32),
    mesh=_mesh,
    scratch_types=[
        pltpu.VMEM((TH, TILE), jnp.int32),
        pltpu.VMEM((TH, TILE), jnp.int32),
        pltpu.VMEM((TILE, D), jnp.float32),
        pltpu.VMEM((TILE, D), jnp.float32),
        pltpu.VMEM_SHARED((R, D), jnp.float32),
        pltpu.SemaphoreType.DMA,
        pltpu.SemaphoreType.DMA,
        pltpu.SemaphoreType.DMA,
    ],
)
def _agg_kernel(g_hbm, src_hbm, dst_hbm, zeros2_hbm, out_hbm,
                src_v, dst_v, buf0, buf1, acc_sh, sm0, sm1, ssem):
    c = lax.axis_index("c")
    s = lax.axis_index("s")
    wbase = (c * NS + s) * TW

    d_z = pltpu.async_copy(zeros2_hbm, acc_sh.at[pl.ds(s * RPS, RPS)], ssem)
    for half in range(2):
        base = wbase + half * TH
        d_src = pltpu.async_copy(src_hbm.at[pl.ds(base, TH)], src_v, sm0)
        d_dst = pltpu.async_copy(dst_hbm.at[pl.ds(base, TH)], dst_v, sm1)
        d_src.wait()
        d_dst.wait()
        if half == 0:
            d_z.wait()
            plsc.subcore_barrier()

        @pl.loop(0, TH, step=NB)
        def _(j):
            g0 = pltpu.async_copy(g_hbm.at[src_v.at[j]], buf0, sm0)
            g1 = pltpu.async_copy(g_hbm.at[src_v.at[j + 1]], buf1, sm1)
            g0.wait()
            s0 = pltpu.async_copy(buf0, acc_sh.at[dst_v.at[j]], ssem, add=True)
            g1.wait()
            s1 = pltpu.async_copy(buf1, acc_sh.at[dst_v.at[j + 1]], ssem,
                                  add=True)
            s0.wait()
            s1.wait()

    plsc.subcore_barrier()
    pltpu.sync_copy(acc_sh.at[pl.ds(s * RPS, RPS)],
                    out_hbm.at[c, pl.ds(s * RPS, RPS)])


_HI = lax.Precision.HIGHEST


def _mm0_body(x1_ref, x2_ref, w_ref, h1_ref, h2_ref):
    h1_ref[...] = jnp.dot(x1_ref[...], w_ref[...],
                          preferred_element_type=jnp.float32, precision=_HI)
    h2_ref[...] = jnp.dot(x2_ref[...], w_ref[...],
                          preferred_element_type=jnp.float32, precision=_HI)


def _dinv_of(dpa_ref, dpb_ref):
    deg = dpa_ref[0:N, :] + dpb_ref[0:N, :] + 1.0
    return lax.rsqrt(jnp.maximum(deg, 1e-12))


def _acc_sum(acc_ref):
    return acc_ref[0, 0:N, :] + acc_ref[1, 0:N, :]


def _scale_body(dpa_ref, dpb_ref, h_ref, g_ref):
    g_ref[...] = h_ref[...] * _dinv_of(dpa_ref, dpb_ref)


def _layer2_body(dpa_ref, dpb_ref, acc_ref, g1_ref, w2_ref, b1_ref, g2_ref):
    dinv = _dinv_of(dpa_ref, dpb_ref)
    pre = (_acc_sum(acc_ref) + g1_ref[...]) * dinv + b1_ref[...]
    a = jnp.maximum(pre, 0.0)
    h2 = jnp.dot(a, w2_ref[...],
                 preferred_element_type=jnp.float32, precision=_HI)
    g2_ref[...] = h2 * dinv


def _znorm_body(dpa_ref, dpb_ref, acc_ref, g2_ref, b2_ref, z_ref):
    dinv = _dinv_of(dpa_ref, dpb_ref)
    o = (_acc_sum(acc_ref) + g2_ref[...]) * dinv + b2_ref[...]
    mu = jnp.mean(o, axis=0, keepdims=True)
    cen = o - mu
    var = jnp.sum(cen * cen, axis=0, keepdims=True) * (1.0 / (N - 1))
    z_ref[...] = cen * lax.rsqrt(var)


_f32 = lambda *shape: jax.ShapeDtypeStruct(shape, jnp.float32)

_mm0 = pl.pallas_call(_mm0_body, out_shape=(_f32(N, D), _f32(N, D)))
_scale = pl.pallas_call(_scale_body, out_shape=_f32(N, D))
_layer2 = pl.pallas_call(_layer2_body, out_shape=_f32(N, D))
_znorm = pl.pallas_call(_znorm_body, out_shape=_f32(N, D))


def _edge_tiles(edge_index):
    pad = EP - E
    src = jnp.concatenate(
        [edge_index[0], jnp.zeros((pad,), jnp.int32)]).reshape(ROWS_T, TILE)
    dst = jnp.concatenate(
        [edge_index[1], jnp.full((pad,), DUMMY, jnp.int32)]).reshape(ROWS_T, TILE)
    return src, dst


def kernel(x1, edge_index1, x2, edge_index2, W1, b1, W2, b2):
    src1, dst1 = _edge_tiles(edge_index1)
    src2, dst2 = _edge_tiles(edge_index2)
    b1r = b1.reshape(1, D)
    b2r = b2.reshape(1, D)

    zeros1 = jnp.zeros((RPS,), jnp.float32)
    zeros2 = jnp.zeros((RPS, D), jnp.float32)
    ones = jnp.ones((TILE,), jnp.float32)

    d1c0, d1c1, d2c0, d2c1 = _deg_kernel(dst1, dst2, zeros1, ones)
    dp1 = (d1c0.reshape(R, 1), d1c1.reshape(R, 1))
    dp2 = (d2c0.reshape(R, 1), d2c1.reshape(R, 1))

    h11, h12 = _mm0(x1, x2, W1)

    zs = []
    for dp, h1, src, dst in ((dp1, h11, src1, dst1), (dp2, h12, src2, dst2)):
        g1 = _scale(*dp, h1)
        acc1 = _agg_kernel(g1, src, dst, zeros2)
        g2 = _layer2(*dp, acc1, g1, W2, b1r)
        acc2 = _agg_kernel(g2, src, dst, zeros2)
        zs.append(_znorm(*dp, acc2, g2, b2r))
    return (zs[0], zs[1])


# speedup vs baseline: 21.3289x; 3.1064x over previous
import functools

import jax
import jax.numpy as jnp
from jax import lax
from jax.experimental import pallas as pl
from jax.experimental.pallas import tpu as pltpu
from jax.experimental.pallas import tpu_sc as plsc

N = 10000
E = 320000
D = 128
NC, NS, L = 2, 16, 16
NW = NC * NS
TILE = 128
TW = 80
EP = NW * TW * TILE
ROWS_T = EP // TILE
R = 10240
RPS = R // NS
DUMMY = R - 1

_mesh = plsc.VectorSubcoreMesh(
    core_axis_name="c", subcore_axis_name="s", num_cores=NC, num_subcores=NS)


_DEG_OUT = tuple(jax.ShapeDtypeStruct((R,), jnp.float32) for _ in range(4))


@functools.partial(
    pl.kernel,
    out_type=_DEG_OUT,
    mesh=_mesh,
    scratch_types=[
        pltpu.VMEM((TW, TILE), jnp.int32),
        pltpu.VMEM((TILE,), jnp.float32),
        pltpu.VMEM_SHARED((R,), jnp.float32),
        pltpu.VMEM_SHARED((R,), jnp.float32),
    ],
)
def _deg_kernel(dst1_hbm, dst2_hbm, zeros1_hbm, ones_hbm,
                o1c0, o1c1, o2c0, o2c1,
                idx_v, ones_v, acc1_sh, acc2_sh):
    c = lax.axis_index("c")
    s = lax.axis_index("s")
    wbase = (c * NS + s) * TW

    pltpu.sync_copy(ones_hbm, ones_v)
    pltpu.sync_copy(zeros1_hbm, acc1_sh.at[pl.ds(s * RPS, RPS)])
    pltpu.sync_copy(zeros1_hbm, acc2_sh.at[pl.ds(s * RPS, RPS)])

    plsc.subcore_barrier()

    for dst_hbm, acc_sh in ((dst1_hbm, acc1_sh), (dst2_hbm, acc2_sh)):
        pltpu.sync_copy(dst_hbm.at[pl.ds(wbase, TW)], idx_v)

        @pl.loop(0, TW)
        def _(j):
            pltpu.sync_copy(ones_v, acc_sh.at[idx_v.at[j]], add=True)

    plsc.subcore_barrier()
    sl = pl.ds(s * RPS, RPS)

    @pl.when(c == 0)
    def _():
        pltpu.sync_copy(acc1_sh.at[sl], o1c0.at[sl])
        pltpu.sync_copy(acc2_sh.at[sl], o2c0.at[sl])

    @pl.when(c == 1)
    def _():
        pltpu.sync_copy(acc1_sh.at[sl], o1c1.at[sl])
        pltpu.sync_copy(acc2_sh.at[sl], o2c1.at[sl])


NB = 2
TH = TW // 2


@functools.partial(
    pl.kernel,
    out_type=jax.ShapeDtypeStruct((NC, R, D), jnp.float32),
    mesh=_mesh,
    scratch_types=[
        pltpu.VMEM((TH, TILE), jnp.int32),
        pltpu.VMEM((TH, TILE), jnp.int32),
        pltpu.VMEM((TILE, D), jnp.float32),
        pltpu.VMEM((TILE, D), jnp.float32),
        pltpu.VMEM_SHARED((R, D), jnp.float32),
        pltpu.SemaphoreType.DMA,
        pltpu.SemaphoreType.DMA,
        pltpu.SemaphoreType.DMA,
    ],
)
def _agg_kernel(g_hbm, src_hbm, dst_hbm, zeros2_hbm, out_hbm,
                src_v, dst_v, buf0, buf1, acc_sh, sm0, sm1, ssem):
    c = lax.axis_index("c")
    s = lax.axis_index("s")
    wbase = (c * NS + s) * TW

    d_z = pltpu.async_copy(zeros2_hbm, acc_sh.at[pl.ds(s * RPS, RPS)], ssem)
    for half in range(2):
        base = wbase + half * TH
        d_src = pltpu.async_copy(src_hbm.at[pl.ds(base, TH)], src_v, sm0)
        d_dst = pltpu.async_copy(dst_hbm.at[pl.ds(base, TH)], dst_v, sm1)
        d_src.wait()
        d_dst.wait()
        if half == 0:
            d_z.wait()
            plsc.subcore_barrier()

        @pl.loop(0, TH, step=NB)
        def _(j):
            g0 = pltpu.async_copy(g_hbm.at[src_v.at[j]], buf0, sm0)
            g1 = pltpu.async_copy(g_hbm.at[src_v.at[j + 1]], buf1, sm1)
            g0.wait()
            s0 = pltpu.async_copy(buf0, acc_sh.at[dst_v.at[j]], ssem, add=True)
            g1.wait()
            s1 = pltpu.async_copy(buf1, acc_sh.at[dst_v.at[j + 1]], ssem,
                                  add=True)
            s0.wait()
            s1.wait()

    plsc.subcore_barrier()
    pltpu.sync_copy(acc_sh.at[pl.ds(s * RPS, RPS)],
                    out_hbm.at[c, pl.ds(s * RPS, RPS)])


_HI = lax.Precision.HIGHEST


def _mm0_body(x1_ref, x2_ref, w_ref, h1_ref, h2_ref):
    h1_ref[...] = jnp.dot(x1_ref[...], w_ref[...],
                          preferred_element_type=jnp.float32, precision=_HI)
    h2_ref[...] = jnp.dot(x2_ref[...], w_ref[...],
                          preferred_element_type=jnp.float32, precision=_HI)


def _dinv_of(dpa_ref, dpb_ref):
    deg = dpa_ref[0:N, :] + dpb_ref[0:N, :] + 1.0
    return lax.rsqrt(jnp.maximum(deg, 1e-12))


def _acc_sum(acc_ref):
    return acc_ref[0, 0:N, :] + acc_ref[1, 0:N, :]


def _scale_body(dpa_ref, dpb_ref, h_ref, g_ref):
    g_ref[...] = h_ref[...] * _dinv_of(dpa_ref, dpb_ref)


def _layer2_body(dpa_ref, dpb_ref, acc_ref, g1_ref, w2_ref, b1_ref, g2_ref):
    dinv = _dinv_of(dpa_ref, dpb_ref)
    pre = (_acc_sum(acc_ref) + g1_ref[...]) * dinv + b1_ref[...]
    a = jnp.maximum(pre, 0.0)
    h2 = jnp.dot(a, w2_ref[...],
                 preferred_element_type=jnp.float32, precision=_HI)
    g2_ref[...] = h2 * dinv


def _znorm_body(dpa_ref, dpb_ref, acc_ref, g2_ref, b2_ref, z_ref):
    dinv = _dinv_of(dpa_ref, dpb_ref)
    o = (_acc_sum(acc_ref) + g2_ref[...]) * dinv + b2_ref[...]
    mu = jnp.mean(o, axis=0, keepdims=True)
    cen = o - mu
    var = jnp.sum(cen * cen, axis=0, keepdims=True) * (1.0 / (N - 1))
    z_ref[...] = cen * lax.rsqrt(var)


_f32 = lambda *shape: jax.ShapeDtypeStruct(shape, jnp.float32)

_mm0 = pl.pallas_call(_mm0_body, out_shape=(_f32(N, D), _f32(N, D)))
_scale = pl.pallas_call(_scale_body, out_shape=_f32(N, D))
_layer2 = pl.pallas_call(_layer2_body, out_shape=_f32(N, D))
_znorm = pl.pallas_call(_znorm_body, out_shape=_f32(N, D))


def _edge_tiles(edge_index):
    pad = EP - E
    pad_i = lax.iota(jnp.int32, pad)
    src = jnp.concatenate(
        [edge_index[0], pad_i % N]).reshape(ROWS_T, TILE)
    dst = jnp.concatenate(
        [edge_index[1], N + pad_i % (R - N)]).reshape(ROWS_T, TILE)
    return src, dst


def kernel(x1, edge_index1, x2, edge_index2, W1, b1, W2, b2):
    src1, dst1 = _edge_tiles(edge_index1)
    src2, dst2 = _edge_tiles(edge_index2)
    b1r = b1.reshape(1, D)
    b2r = b2.reshape(1, D)

    zeros1 = jnp.zeros((RPS,), jnp.float32)
    zeros2 = jnp.zeros((RPS, D), jnp.float32)
    ones = jnp.ones((TILE,), jnp.float32)

    d1c0, d1c1, d2c0, d2c1 = _deg_kernel(dst1, dst2, zeros1, ones)
    dp1 = (d1c0.reshape(R, 1), d1c1.reshape(R, 1))
    dp2 = (d2c0.reshape(R, 1), d2c1.reshape(R, 1))

    h11, h12 = _mm0(x1, x2, W1)

    zs = []
    for dp, h1, src, dst in ((dp1, h11, src1, dst1), (dp2, h12, src2, dst2)):
        g1 = _scale(*dp, h1)
        acc1 = _agg_kernel(g1, src, dst, zeros2)
        g2 = _layer2(*dp, acc1, g1, W2, b1r)
        acc2 = _agg_kernel(g2, src, dst, zeros2)
        zs.append(_znorm(*dp, acc2, g2, b2r))
    return (zs[0], zs[1])


# speedup vs baseline: 21.5232x; 1.0091x over previous
import functools

import jax
import jax.numpy as jnp
from jax import lax
from jax.experimental import pallas as pl
from jax.experimental.pallas import tpu as pltpu
from jax.experimental.pallas import tpu_sc as plsc

N = 10000
E = 320000
D = 128
NC, NS, L = 2, 16, 16
NW = NC * NS
TILE = 128
TW = 80
EP = NW * TW * TILE
ROWS_T = EP // TILE
R = 10240
RPS = R // NS
DUMMY = R - 1

_mesh = plsc.VectorSubcoreMesh(
    core_axis_name="c", subcore_axis_name="s", num_cores=NC, num_subcores=NS)


_DEG_OUT = tuple(jax.ShapeDtypeStruct((R,), jnp.float32) for _ in range(4))


@functools.partial(
    pl.kernel,
    out_type=_DEG_OUT,
    mesh=_mesh,
    scratch_types=[
        pltpu.VMEM((TW, TILE), jnp.int32),
        pltpu.VMEM((TILE,), jnp.float32),
        pltpu.VMEM_SHARED((R,), jnp.float32),
        pltpu.VMEM_SHARED((R,), jnp.float32),
        pltpu.SemaphoreType.DMA,
        pltpu.SemaphoreType.DMA,
    ],
)
def _deg_kernel(dst1_hbm, dst2_hbm, zeros1_hbm, ones_hbm,
                o1c0, o1c1, o2c0, o2c1,
                idx_v, ones_v, acc1_sh, acc2_sh, sm0, sm1):
    c = lax.axis_index("c")
    s = lax.axis_index("s")
    wbase = (c * NS + s) * TW

    pltpu.sync_copy(ones_hbm, ones_v)
    pltpu.sync_copy(zeros1_hbm, acc1_sh.at[pl.ds(s * RPS, RPS)])
    pltpu.sync_copy(zeros1_hbm, acc2_sh.at[pl.ds(s * RPS, RPS)])

    plsc.subcore_barrier()

    for dst_hbm, acc_sh in ((dst1_hbm, acc1_sh), (dst2_hbm, acc2_sh)):
        pltpu.sync_copy(dst_hbm.at[pl.ds(wbase, TW)], idx_v)

        @pl.loop(0, TW, step=2)
        def _(j):
            s0 = pltpu.async_copy(ones_v, acc_sh.at[idx_v.at[j]], sm0,
                                  add=True)
            s1 = pltpu.async_copy(ones_v, acc_sh.at[idx_v.at[j + 1]], sm1,
                                  add=True)
            s0.wait()
            s1.wait()

    plsc.subcore_barrier()
    sl = pl.ds(s * RPS, RPS)

    @pl.when(c == 0)
    def _():
        pltpu.sync_copy(acc1_sh.at[sl], o1c0.at[sl])
        pltpu.sync_copy(acc2_sh.at[sl], o2c0.at[sl])

    @pl.when(c == 1)
    def _():
        pltpu.sync_copy(acc1_sh.at[sl], o1c1.at[sl])
        pltpu.sync_copy(acc2_sh.at[sl], o2c1.at[sl])


NB = 2
TH = TW // 2


@functools.partial(
    pl.kernel,
    out_type=jax.ShapeDtypeStruct((NC, R, D), jnp.float32),
    mesh=_mesh,
    scratch_types=[
        pltpu.VMEM((TH, TILE), jnp.int32),
        pltpu.VMEM((TH, TILE), jnp.int32),
        pltpu.VMEM((TILE, D), jnp.float32),
        pltpu.VMEM((TILE, D), jnp.float32),
        pltpu.VMEM_SHARED((R, D), jnp.float32),
        pltpu.SemaphoreType.DMA,
        pltpu.SemaphoreType.DMA,
        pltpu.SemaphoreType.DMA,
    ],
)
def _agg_kernel(g_hbm, src_hbm, dst_hbm, zeros2_hbm, out_hbm,
                src_v, dst_v, buf0, buf1, acc_sh, sm0, sm1, ssem):
    c = lax.axis_index("c")
    s = lax.axis_index("s")
    wbase = (c * NS + s) * TW

    d_z = pltpu.async_copy(zeros2_hbm, acc_sh.at[pl.ds(s * RPS, RPS)], ssem)
    for half in range(2):
        base = wbase + half * TH
        d_src = pltpu.async_copy(src_hbm.at[pl.ds(base, TH)], src_v, sm0)
        d_dst = pltpu.async_copy(dst_hbm.at[pl.ds(base, TH)], dst_v, sm1)
        d_src.wait()
        d_dst.wait()
        if half == 0:
            d_z.wait()
            plsc.subcore_barrier()

        @pl.loop(0, TH, step=NB)
        def _(j):
            g0 = pltpu.async_copy(g_hbm.at[src_v.at[j]], buf0, sm0)
            g1 = pltpu.async_copy(g_hbm.at[src_v.at[j + 1]], buf1, sm1)
            g0.wait()
            s0 = pltpu.async_copy(buf0, acc_sh.at[dst_v.at[j]], ssem, add=True)
            g1.wait()
            s1 = pltpu.async_copy(buf1, acc_sh.at[dst_v.at[j + 1]], ssem,
                                  add=True)
            s0.wait()
            s1.wait()

    plsc.subcore_barrier()
    pltpu.sync_copy(acc_sh.at[pl.ds(s * RPS, RPS)],
                    out_hbm.at[c, pl.ds(s * RPS, RPS)])


_HI = lax.Precision.HIGHEST


def _mm0_body(x1_ref, x2_ref, w_ref, h1_ref, h2_ref):
    h1_ref[...] = jnp.dot(x1_ref[...], w_ref[...],
                          preferred_element_type=jnp.float32, precision=_HI)
    h2_ref[...] = jnp.dot(x2_ref[...], w_ref[...],
                          preferred_element_type=jnp.float32, precision=_HI)


def _dinv_of(dpa_ref, dpb_ref):
    deg = dpa_ref[0:N, :] + dpb_ref[0:N, :] + 1.0
    return lax.rsqrt(jnp.maximum(deg, 1e-12))


def _acc_sum(acc_ref):
    return acc_ref[0, 0:N, :] + acc_ref[1, 0:N, :]


def _scale_body(dpa_ref, dpb_ref, h_ref, g_ref):
    g_ref[...] = h_ref[...] * _dinv_of(dpa_ref, dpb_ref)


def _layer2_body(dpa_ref, dpb_ref, acc_ref, g1_ref, w2_ref, b1_ref, g2_ref):
    dinv = _dinv_of(dpa_ref, dpb_ref)
    pre = (_acc_sum(acc_ref) + g1_ref[...]) * dinv + b1_ref[...]
    a = jnp.maximum(pre, 0.0)
    h2 = jnp.dot(a, w2_ref[...],
                 preferred_element_type=jnp.float32, precision=_HI)
    g2_ref[...] = h2 * dinv


def _znorm_body(dpa_ref, dpb_ref, acc_ref, g2_ref, b2_ref, z_ref):
    dinv = _dinv_of(dpa_ref, dpb_ref)
    o = (_acc_sum(acc_ref) + g2_ref[...]) * dinv + b2_ref[...]
    mu = jnp.mean(o, axis=0, keepdims=True)
    cen = o - mu
    var = jnp.sum(cen * cen, axis=0, keepdims=True) * (1.0 / (N - 1))
    z_ref[...] = cen * lax.rsqrt(var)


_f32 = lambda *shape: jax.ShapeDtypeStruct(shape, jnp.float32)

_mm0 = pl.pallas_call(_mm0_body, out_shape=(_f32(N, D), _f32(N, D)))
_scale = pl.pallas_call(_scale_body, out_shape=_f32(N, D))
_layer2 = pl.pallas_call(_layer2_body, out_shape=_f32(N, D))
_znorm = pl.pallas_call(_znorm_body, out_shape=_f32(N, D))


def _edge_tiles(edge_index):
    pad = EP - E
    pad_i = lax.iota(jnp.int32, pad)
    src = jnp.concatenate(
        [edge_index[0], pad_i % N]).reshape(ROWS_T, TILE)
    dst = jnp.concatenate(
        [edge_index[1], N + pad_i % (R - N)]).reshape(ROWS_T, TILE)
    return src, dst


def kernel(x1, edge_index1, x2, edge_index2, W1, b1, W2, b2):
    src1, dst1 = _edge_tiles(edge_index1)
    src2, dst2 = _edge_tiles(edge_index2)
    b1r = b1.reshape(1, D)
    b2r = b2.reshape(1, D)

    zeros1 = jnp.zeros((RPS,), jnp.float32)
    zeros2 = jnp.zeros((RPS, D), jnp.float32)
    ones = jnp.ones((TILE,), jnp.float32)

    d1c0, d1c1, d2c0, d2c1 = _deg_kernel(dst1, dst2, zeros1, ones)
    dp1 = (d1c0.reshape(R, 1), d1c1.reshape(R, 1))
    dp2 = (d2c0.reshape(R, 1), d2c1.reshape(R, 1))

    h11, h12 = _mm0(x1, x2, W1)

    zs = []
    for dp, h1, src, dst in ((dp1, h11, src1, dst1), (dp2, h12, src2, dst2)):
        g1 = _scale(*dp, h1)
        acc1 = _agg_kernel(g1, src, dst, zeros2)
        g2 = _layer2(*dp, acc1, g1, W2, b1r)
        acc2 = _agg_kernel(g2, src, dst, zeros2)
        zs.append(_znorm(*dp, acc2, g2, b2r))
    return (zs[0], zs[1])
